# baseline (speedup 1.0000x reference)
import functools

import jax
import jax.numpy as jnp
import numpy as np
from jax import lax
from jax.experimental import pallas as pl
from jax.experimental.pallas import tpu as pltpu
from jax.experimental.pallas import tpu_sc as plsc

N = 100000
C = 480
G = 64
CS = 128
EPS = 1e-05
CH = 32
NCH = N // CH
NC, NS = 2, 16
NW = NC * NS
NPAIR = (NCH + 2 * NW - 1) // (2 * NW)
NTRI = (NCH + 3 * NW - 1) // (3 * NW)
L = 16

_f32 = jnp.float32
_i32 = jnp.int32


def _wait(src, dst, sem):
    pltpu.make_async_copy(src, dst, sem).wait()


@functools.partial(
    pl.kernel,
    out_type=[
        jax.ShapeDtypeStruct((NW, G, C), _f32),
        jax.ShapeDtypeStruct((NW, G, CS), _f32),
        jax.ShapeDtypeStruct((NW, G, L), _f32),
    ],
    mesh=plsc.VectorSubcoreMesh(core_axis_name="c", subcore_axis_name="s"),
    scratch_types=[
        pltpu.VMEM((CH, C), _f32),
        pltpu.VMEM((CH, C), _f32),
        pltpu.VMEM((CH + L,), _i32),
        pltpu.VMEM((CH + L,), _i32),
        pltpu.VMEM((G, C), _f32),
        pltpu.VMEM((G, CS), _f32),
        pltpu.VMEM((G, L), _f32),
        pltpu.SemaphoreType.DMA,
        pltpu.SemaphoreType.DMA,
        pltpu.SemaphoreType.DMA,
        pltpu.SemaphoreType.DMA,
    ],
)
def _stats_kernel(x_hbm, batch_hbm, sums_hbm, sq_hbm, cnt_hbm,
                  xb0, xb1, ix0, ix1, acc, acc2, cnt,
                  ld0, ld1, ldi0, ldi1):
    cid = lax.axis_index("c")
    sid = lax.axis_index("s")
    wid = cid * NS + sid

    zero = jnp.zeros((L,), _f32)
    one = jnp.ones((L,), _f32)

    def _zero(g, carry):
        for j in range(C // L):
            acc[g, pl.ds(j * L, L)] = zero
        for j in range(CS // L):
            acc2[g, pl.ds(j * L, L)] = zero
        cnt[g, pl.ds(0, L)] = zero
        return carry
    lax.fori_loop(0, G, _zero, 0)

    bufs = ((xb0, ix0, ld0, ldi0), (xb1, ix1, ld1, ldi1))

    def _start_load(c, xb, ix, ld, ldi):
        pltpu.async_copy(x_hbm.at[pl.ds(c * CH, CH)], xb, ld)
        pltpu.async_copy(batch_hbm.at[pl.ds(c * CH, CH)], ix.at[pl.ds(0, CH)], ldi)

    for b in range(2):
        xb, ix, ld, ldi = bufs[b]
        _start_load(wid + b * NW, xb, ix, ld, ldi)

    def _pair(i, carry):
        for b in range(2):
            xb, ix, ld, ldi = bufs[b]
            c = wid + (2 * i + b) * NW

            @pl.when(c < NCH)
            def _():
                _wait(x_hbm.at[pl.ds(0, CH)], xb, ld)
                _wait(batch_hbm.at[pl.ds(0, CH)], ix.at[pl.ds(0, CH)], ldi)

                g0 = ix[pl.ds(0, L)][0]
                gl = ix[pl.ds(CH - 1, L)][0]

                @pl.when(g0 == gl)
                def _():
                    for jb in range(2):
                        js = [jb * 4 + k for k in range(4)]

                        def _rowA(r, carry):
                            s, q = carry
                            s2, q2 = [], []
                            for k, j in enumerate(js):
                                v = xb[r, pl.ds(j * L, L)]
                                s2.append(s[k] + v)
                                q2.append(q[k] + v * v)
                            return tuple(s2), tuple(q2)
                        s, q = lax.fori_loop(
                            0, CH, _rowA,
                            (tuple(zero for _ in js), tuple(zero for _ in js)))
                        for k, j in enumerate(js):
                            plsc.addupdate(acc.at[g0, pl.ds(j * L, L)], s[k])
                            plsc.addupdate(acc2.at[g0, pl.ds(j * L, L)], q[k])
                    for js in ([8, 9, 10, 11, 12], [13, 14, 15, 16, 17],
                               [18, 19, 20, 21, 22], [23, 24, 25, 26, 27],
                               [28, 29]):
                        def _rowB(r, s):
                            return tuple(
                                s[k] + xb[r, pl.ds(j * L, L)]
                                for k, j in enumerate(js))
                        s = lax.fori_loop(0, CH, _rowB,
                                          tuple(zero for _ in js))
                        for k, j in enumerate(js):
                            plsc.addupdate(acc.at[g0, pl.ds(j * L, L)], s[k])
                    plsc.addupdate(cnt.at[g0, pl.ds(0, L)], one * float(CH))

                @pl.when(g0 != gl)
                def _():
                    def _row(r, c2):
                        g = ix[pl.ds(r, L)][0]
                        for j in range(C // L):
                            v = xb[r, pl.ds(j * L, L)]
                            plsc.addupdate(acc.at[g, pl.ds(j * L, L)], v)
                            if j < CS // L:
                                plsc.addupdate(acc2.at[g, pl.ds(j * L, L)], v * v)
                        plsc.addupdate(cnt.at[g, pl.ds(0, L)], one)
                        return c2
                    lax.fori_loop(0, CH, _row, 0)

                cn = c + 2 * NW

                @pl.when(cn < NCH)
                def _():
                    _start_load(cn, xb, ix, ld, ldi)
        return carry

    lax.fori_loop(0, NPAIR, _pair, 0)

    pltpu.sync_copy(acc, sums_hbm.at[wid])
    pltpu.sync_copy(acc2, sq_hbm.at[wid])
    pltpu.sync_copy(cnt, cnt_hbm.at[wid])


@functools.partial(
    pl.kernel,
    out_type=jax.ShapeDtypeStruct((N, C), _f32),
    mesh=plsc.VectorSubcoreMesh(core_axis_name="c", subcore_axis_name="s"),
    scratch_types=[
        pltpu.VMEM((CH, C), _f32),
        pltpu.VMEM((CH, C), _f32),
        pltpu.VMEM((CH, C), _f32),
        pltpu.VMEM((CH + L,), _i32),
        pltpu.VMEM((CH + L,), _i32),
        pltpu.VMEM((CH + L,), _i32),
        pltpu.VMEM((G, C), _f32),
        pltpu.VMEM((G, C), _f32),
        pltpu.SemaphoreType.DMA,
        pltpu.SemaphoreType.DMA,
        pltpu.SemaphoreType.DMA,
        pltpu.SemaphoreType.DMA,
        pltpu.SemaphoreType.DMA,
        pltpu.SemaphoreType.DMA,
        pltpu.SemaphoreType.DMA,
        pltpu.SemaphoreType.DMA,
        pltpu.SemaphoreType.DMA,
    ],
)
def _apply_kernel(x_hbm, batch_hbm, alpha_hbm, beta_hbm, out_hbm,
                  xb0, xb1, xb2, ix0, ix1, ix2, al, be,
                  ld0, ld1, ld2, ldi0, ldi1, ldi2, st0, st1, st2):
    cid = lax.axis_index("c")
    sid = lax.axis_index("s")
    wid = cid * NS + sid

    pltpu.sync_copy(alpha_hbm, al)
    pltpu.sync_copy(beta_hbm, be)

    bufs = ((xb0, ix0, ld0, ldi0, st0),
            (xb1, ix1, ld1, ldi1, st1),
            (xb2, ix2, ld2, ldi2, st2))

    def _start_load(c, xb, ix, ld, ldi):
        pltpu.async_copy(x_hbm.at[pl.ds(c * CH, CH)], xb, ld)
        pltpu.async_copy(batch_hbm.at[pl.ds(c * CH, CH)], ix.at[pl.ds(0, CH)], ldi)

    for b in range(3):
        xb, ix, ld, ldi, st = bufs[b]
        _start_load(wid + b * NW, xb, ix, ld, ldi)

    def _tri(i, carry):
        for k in range(3):
            xb, ix, ld, ldi, st = bufs[k]
            xp, ixp, ldp, ldip, stp = bufs[(k + 2) % 3]
            t = 3 * i + k
            c = wid + t * NW

            @pl.when(c < NCH)
            def _():
                _wait(x_hbm.at[pl.ds(0, CH)], xb, ld)
                _wait(batch_hbm.at[pl.ds(0, CH)], ix.at[pl.ds(0, CH)], ldi)

                g0 = ix[pl.ds(0, L)][0]
                gl = ix[pl.ds(CH - 1, L)][0]

                @pl.when(g0 == gl)
                def _():
                    for jb in range(6):
                        js = [jb * 5 + k for k in range(5)]
                        va = [al[g0, pl.ds(j * L, L)] for j in js]
                        vb = [be[g0, pl.ds(j * L, L)] for j in js]

                        def _rowF(r, c2):
                            for k, j in enumerate(js):
                                sl = pl.ds(j * L, L)
                                xb[r, sl] = xb[r, sl] * va[k] + vb[k]
                            return c2
                        lax.fori_loop(0, CH, _rowF, 0)

                @pl.when(g0 != gl)
                def _():
                    def _row(r, c2):
                        g = ix[pl.ds(r, L)][0]
                        for j in range(C // L):
                            sl = pl.ds(j * L, L)
                            xb[r, sl] = xb[r, sl] * al[g, sl] + be[g, sl]
                        return c2
                    lax.fori_loop(0, CH, _row, 0)

                pltpu.async_copy(xb, out_hbm.at[pl.ds(c * CH, CH)], st)

            cl = c + 2 * NW

            @pl.when((t >= 1) & (cl < NCH))
            def _():
                _wait(xp, out_hbm.at[pl.ds(0, CH)], stp)
                _start_load(cl, xp, ixp, ldp, ldip)
        return carry

    lax.fori_loop(0, NTRI, _tri, 0)

    _wait(xb0, out_hbm.at[pl.ds(0, CH)], st0)
    _wait(xb1, out_hbm.at[pl.ds(0, CH)], st1)
    _wait(xb2, out_hbm.at[pl.ds(0, CH)], st2)


BLK = 1000
NBLK = N // BLK


@functools.partial(
    pl.pallas_call,
    grid=(NBLK,),
    in_specs=[
        pl.BlockSpec((BLK, C), lambda i: (i, 0)),
        pl.BlockSpec((1, 1, BLK), lambda i: (i, 0, 0)),
        pl.BlockSpec((G, C), lambda i: (0, 0)),
        pl.BlockSpec((G, C), lambda i: (0, 0)),
    ],
    out_specs=pl.BlockSpec((BLK, C), lambda i: (i, 0)),
    out_shape=jax.ShapeDtypeStruct((N, C), _f32),
)
def _apply_tc(x_ref, b_ref, al_ref, be_ref, o_ref):
    bblk = b_ref[0, 0, :]
    oh = (bblk[:, None]
          == jax.lax.broadcasted_iota(_i32, (BLK, G), 1)).astype(_f32)
    a = jax.lax.dot(oh, al_ref[...], precision=jax.lax.Precision.HIGHEST)
    b = jax.lax.dot(oh, be_ref[...], precision=jax.lax.Precision.HIGHEST)
    o_ref[...] = x_ref[...] * a + b


_P_np = np.zeros((C, 16), np.float32)
_SELT_np = np.zeros((16, C), np.float32)
_P_np[:CS, 0] = 1.0 / CS
_SELT_np[0, :CS] = 1.0
for _c in range(128, 320):
    _d = (_c - 128) % 3
    _P_np[_c, 1 + _d] = 1.0 / 64.0
    _SELT_np[1 + _d, _c] = 1.0
for _c in range(320, 480):
    _d = (_c - 320) % 5
    _P_np[_c, 4 + _d] = 1.0 / 32.0
    _SELT_np[4 + _d, _c] = 1.0
_MASKS_np = np.concatenate([np.ones(CS, np.float32),
                            np.zeros(C - CS, np.float32)])


@functools.partial(
    pl.pallas_call,
    out_shape=[
        jax.ShapeDtypeStruct((G, C), _f32),
        jax.ShapeDtypeStruct((G, C), _f32),
    ],
)
def _tables_kernel(sums_ref, sq_ref, cnt_ref, wcol_ref, bcol_ref,
                   p_ref, selt_ref, masks_ref,
                   alpha_ref, beta_ref):
    S = jnp.sum(sums_ref[...], axis=0)
    q = jnp.sum(sq_ref[...], axis=(0, 2))[:, None]
    n = jnp.sum(cnt_ref[...], axis=0)[:, 0:1]
    nc = jnp.maximum(n, 1.0)

    stats = jax.lax.dot(S, p_ref[...],
                        precision=jax.lax.Precision.HIGHEST) / nc
    m = stats[:, 0:1]
    v = jnp.maximum(q / (CS * nc) - m * m, 0.0)
    inv = 1.0 / (jnp.sqrt(v) + EPS)

    sub = jax.lax.dot(stats, selt_ref[...],
                      precision=jax.lax.Precision.HIGHEST)
    masks = masks_ref[...][None, :]
    a = (inv * masks + (1.0 - masks)) * wcol_ref[...][None, :]
    alpha_ref[...] = a
    beta_ref[...] = bcol_ref[...][None, :] - sub * a


def kernel(x, batch, weight, bias):
    batch = batch.astype(_i32)
    sums_p, sq_p, cnt_p = _stats_kernel(x, batch)

    S = sums_p.sum(axis=0)
    Q = sq_p.sum(axis=(0, 2))
    cnt = cnt_p.sum(axis=0)[:, 0]
    cntc = jnp.maximum(cnt, 1.0)

    m = S[:, :CS].sum(axis=1) / (CS * cntc)
    v = jnp.maximum(Q / (CS * cntc) - m * m, 0.0)
    inv = 1.0 / (jnp.sqrt(v) + EPS)
    muT = S[:, 128:320].reshape(G, 64, 3).sum(axis=1) / (64.0 * cntc)[:, None]
    muU = S[:, 320:480].reshape(G, 32, 5).sum(axis=1) / (32.0 * cntc)[:, None]

    w0 = weight[:128]
    w1 = jnp.repeat(weight[128:192], 3)
    w2 = jnp.repeat(weight[192:224], 5)
    alpha = jnp.concatenate([
        inv[:, None] * w0[None, :],
        jnp.broadcast_to(w1[None, :], (G, 192)),
        jnp.broadcast_to(w2[None, :], (G, 160)),
    ], axis=1)
    beta = jnp.concatenate([
        bias[None, :] - (m * inv)[:, None] * w0[None, :],
        -jnp.tile(muT, (1, 64)) * w1[None, :],
        -jnp.tile(muU, (1, 32)) * w2[None, :],
    ], axis=1)

    return _apply_tc(x, batch.reshape(NBLK, 1, BLK), alpha, beta)


# speedup vs baseline: 1.2615x; 1.2615x over previous
import functools

import jax
import jax.numpy as jnp
from jax import lax
from jax.experimental import pallas as pl
from jax.experimental.pallas import tpu as pltpu
from jax.experimental.pallas import tpu_sc as plsc

N = 100000
C = 480
G = 64
CS = 128
EPS = 1e-05
CH = 40
NCH = N // CH
NC, NS = 2, 16
NW = NC * NS
NPAIR = (NCH + 2 * NW - 1) // (2 * NW)
NTRI = (NCH + 3 * NW - 1) // (3 * NW)
L = 16

_f32 = jnp.float32
_i32 = jnp.int32


def _wait(src, dst, sem):
    pltpu.make_async_copy(src, dst, sem).wait()


@functools.partial(
    pl.kernel,
    out_type=[
        jax.ShapeDtypeStruct((NW, G, C), _f32),
        jax.ShapeDtypeStruct((NW, G, CS), _f32),
        jax.ShapeDtypeStruct((NW, G, L), _f32),
    ],
    mesh=plsc.VectorSubcoreMesh(core_axis_name="c", subcore_axis_name="s"),
    scratch_types=[
        pltpu.VMEM((CH, C), _f32),
        pltpu.VMEM((CH, C), _f32),
        pltpu.VMEM((CH + L,), _i32),
        pltpu.VMEM((CH + L,), _i32),
        pltpu.VMEM((G, C), _f32),
        pltpu.VMEM((G, CS), _f32),
        pltpu.VMEM((G, L), _f32),
        pltpu.SemaphoreType.DMA,
        pltpu.SemaphoreType.DMA,
        pltpu.SemaphoreType.DMA,
        pltpu.SemaphoreType.DMA,
    ],
)
def _stats_kernel(x_hbm, batch_hbm, sums_hbm, sq_hbm, cnt_hbm,
                  xb0, xb1, ix0, ix1, acc, acc2, cnt,
                  ld0, ld1, ldi0, ldi1):
    cid = lax.axis_index("c")
    sid = lax.axis_index("s")
    wid = cid * NS + sid

    zero = jnp.zeros((L,), _f32)
    one = jnp.ones((L,), _f32)

    def _zero(g, carry):
        for j in range(C // L):
            acc[g, pl.ds(j * L, L)] = zero
        for j in range(CS // L):
            acc2[g, pl.ds(j * L, L)] = zero
        cnt[g, pl.ds(0, L)] = zero
        return carry
    lax.fori_loop(0, G, _zero, 0)

    bufs = ((xb0, ix0, ld0, ldi0), (xb1, ix1, ld1, ldi1))

    def _start_load(c, xb, ix, ld, ldi):
        pltpu.async_copy(x_hbm.at[pl.ds(c * CH, CH)], xb, ld)
        pltpu.async_copy(batch_hbm.at[pl.ds(c * CH, CH)], ix.at[pl.ds(0, CH)], ldi)

    for b in range(2):
        xb, ix, ld, ldi = bufs[b]
        _start_load(wid + b * NW, xb, ix, ld, ldi)

    def _pair(i, carry):
        for b in range(2):
            xb, ix, ld, ldi = bufs[b]
            c = wid + (2 * i + b) * NW

            @pl.when(c < NCH)
            def _():
                _wait(x_hbm.at[pl.ds(0, CH)], xb, ld)
                _wait(batch_hbm.at[pl.ds(0, CH)], ix.at[pl.ds(0, CH)], ldi)

                g0 = ix[pl.ds(0, L)][0]
                gl = ix[pl.ds(CH - 1, L)][0]

                @pl.when(g0 == gl)
                def _():
                    for jb in range(2):
                        js = [jb * 4 + k for k in range(4)]

                        def _rowA(r, carry):
                            s, q = carry
                            s2, q2 = [], []
                            for k, j in enumerate(js):
                                v = xb[r, pl.ds(j * L, L)]
                                s2.append(s[k] + v)
                                q2.append(q[k] + v * v)
                            return tuple(s2), tuple(q2)
                        s, q = lax.fori_loop(
                            0, CH, _rowA,
                            (tuple(zero for _ in js), tuple(zero for _ in js)))
                        for k, j in enumerate(js):
                            plsc.addupdate(acc.at[g0, pl.ds(j * L, L)], s[k])
                            plsc.addupdate(acc2.at[g0, pl.ds(j * L, L)], q[k])
                    for js in ([8, 9, 10, 11, 12], [13, 14, 15, 16, 17],
                               [18, 19, 20, 21, 22], [23, 24, 25, 26, 27],
                               [28, 29]):
                        def _rowB(r, s):
                            return tuple(
                                s[k] + xb[r, pl.ds(j * L, L)]
                                for k, j in enumerate(js))
                        s = lax.fori_loop(0, CH, _rowB,
                                          tuple(zero for _ in js))
                        for k, j in enumerate(js):
                            plsc.addupdate(acc.at[g0, pl.ds(j * L, L)], s[k])
                    plsc.addupdate(cnt.at[g0, pl.ds(0, L)], one * float(CH))

                @pl.when(g0 != gl)
                def _():
                    def _row(r, c2):
                        g = ix[pl.ds(r, L)][0]
                        for j in range(C // L):
                            v = xb[r, pl.ds(j * L, L)]
                            plsc.addupdate(acc.at[g, pl.ds(j * L, L)], v)
                            if j < CS // L:
                                plsc.addupdate(acc2.at[g, pl.ds(j * L, L)], v * v)
                        plsc.addupdate(cnt.at[g, pl.ds(0, L)], one)
                        return c2
                    lax.fori_loop(0, CH, _row, 0)

                cn = c + 2 * NW

                @pl.when(cn < NCH)
                def _():
                    _start_load(cn, xb, ix, ld, ldi)
        return carry

    lax.fori_loop(0, NPAIR, _pair, 0)

    pltpu.sync_copy(acc, sums_hbm.at[wid])
    pltpu.sync_copy(acc2, sq_hbm.at[wid])
    pltpu.sync_copy(cnt, cnt_hbm.at[wid])


@functools.partial(
    pl.kernel,
    out_type=jax.ShapeDtypeStruct((N, C), _f32),
    mesh=plsc.VectorSubcoreMesh(core_axis_name="c", subcore_axis_name="s"),
    scratch_types=[
        pltpu.VMEM((CH, C), _f32),
        pltpu.VMEM((CH, C), _f32),
        pltpu.VMEM((CH, C), _f32),
        pltpu.VMEM((CH + L,), _i32),
        pltpu.VMEM((CH + L,), _i32),
        pltpu.VMEM((CH + L,), _i32),
        pltpu.VMEM((G, C), _f32),
        pltpu.VMEM((G, C), _f32),
        pltpu.SemaphoreType.DMA,
        pltpu.SemaphoreType.DMA,
        pltpu.SemaphoreType.DMA,
        pltpu.SemaphoreType.DMA,
        pltpu.SemaphoreType.DMA,
        pltpu.SemaphoreType.DMA,
        pltpu.SemaphoreType.DMA,
        pltpu.SemaphoreType.DMA,
        pltpu.SemaphoreType.DMA,
    ],
)
def _apply_kernel(x_hbm, batch_hbm, alpha_hbm, beta_hbm, out_hbm,
                  xb0, xb1, xb2, ix0, ix1, ix2, al, be,
                  ld0, ld1, ld2, ldi0, ldi1, ldi2, st0, st1, st2):
    cid = lax.axis_index("c")
    sid = lax.axis_index("s")
    wid = cid * NS + sid

    pltpu.sync_copy(alpha_hbm, al)
    pltpu.sync_copy(beta_hbm, be)

    bufs = ((xb0, ix0, ld0, ldi0, st0),
            (xb1, ix1, ld1, ldi1, st1),
            (xb2, ix2, ld2, ldi2, st2))

    def _start_load(c, xb, ix, ld, ldi):
        pltpu.async_copy(x_hbm.at[pl.ds(c * CH, CH)], xb, ld)
        pltpu.async_copy(batch_hbm.at[pl.ds(c * CH, CH)], ix.at[pl.ds(0, CH)], ldi)

    for b in range(3):
        xb, ix, ld, ldi, st = bufs[b]
        _start_load(wid + b * NW, xb, ix, ld, ldi)

    def _tri(i, carry):
        for k in range(3):
            xb, ix, ld, ldi, st = bufs[k]
            xp, ixp, ldp, ldip, stp = bufs[(k + 2) % 3]
            t = 3 * i + k
            c = wid + t * NW

            @pl.when(c < NCH)
            def _():
                _wait(x_hbm.at[pl.ds(0, CH)], xb, ld)
                _wait(batch_hbm.at[pl.ds(0, CH)], ix.at[pl.ds(0, CH)], ldi)

                g0 = ix[pl.ds(0, L)][0]
                gl = ix[pl.ds(CH - 1, L)][0]

                @pl.when(g0 == gl)
                def _():
                    for jb in range(6):
                        js = [jb * 5 + k for k in range(5)]
                        va = [al[g0, pl.ds(j * L, L)] for j in js]
                        vb = [be[g0, pl.ds(j * L, L)] for j in js]

                        def _rowF(r, c2):
                            for k, j in enumerate(js):
                                sl = pl.ds(j * L, L)
                                xb[r, sl] = xb[r, sl] * va[k] + vb[k]
                            return c2
                        lax.fori_loop(0, CH, _rowF, 0)

                @pl.when(g0 != gl)
                def _():
                    def _row(r, c2):
                        g = ix[pl.ds(r, L)][0]
                        for j in range(C // L):
                            sl = pl.ds(j * L, L)
                            xb[r, sl] = xb[r, sl] * al[g, sl] + be[g, sl]
                        return c2
                    lax.fori_loop(0, CH, _row, 0)

                pltpu.async_copy(xb, out_hbm.at[pl.ds(c * CH, CH)], st)

            cl = c + 2 * NW

            @pl.when((t >= 1) & (cl < NCH))
            def _():
                _wait(xp, out_hbm.at[pl.ds(0, CH)], stp)
                _start_load(cl, xp, ixp, ldp, ldip)
        return carry

    lax.fori_loop(0, NTRI, _tri, 0)

    _wait(xb0, out_hbm.at[pl.ds(0, CH)], st0)
    _wait(xb1, out_hbm.at[pl.ds(0, CH)], st1)
    _wait(xb2, out_hbm.at[pl.ds(0, CH)], st2)


def kernel(x, batch, weight, bias):
    batch = batch.astype(_i32)
    sums_p, sq_p, cnt_p = _stats_kernel(x, batch)

    S = sums_p.sum(axis=0)
    Q = sq_p.sum(axis=(0, 2))
    cnt = cnt_p.sum(axis=0)[:, 0]
    cntc = jnp.maximum(cnt, 1.0)

    m = S[:, :CS].sum(axis=1) / (CS * cntc)
    v = jnp.maximum(Q / (CS * cntc) - m * m, 0.0)
    inv = 1.0 / (jnp.sqrt(v) + EPS)
    muT = S[:, 128:320].reshape(G, 64, 3).sum(axis=1) / (64.0 * cntc)[:, None]
    muU = S[:, 320:480].reshape(G, 32, 5).sum(axis=1) / (32.0 * cntc)[:, None]

    w0 = weight[:128]
    w1 = jnp.repeat(weight[128:192], 3)
    w2 = jnp.repeat(weight[192:224], 5)
    alpha = jnp.concatenate([
        inv[:, None] * w0[None, :],
        jnp.broadcast_to(w1[None, :], (G, 192)),
        jnp.broadcast_to(w2[None, :], (G, 160)),
    ], axis=1)
    beta = jnp.concatenate([
        bias[None, :] - (m * inv)[:, None] * w0[None, :],
        -jnp.tile(muT, (1, 64)) * w1[None, :],
        -jnp.tile(muU, (1, 32)) * w2[None, :],
    ], axis=1)

    return _apply_kernel(x, batch, alpha, beta)
